# baseline (speedup 1.0000x reference)
import jax
import jax.numpy as jnp
from jax import lax
from jax.experimental import pallas as pl
from jax.experimental.pallas import tpu as pltpu
from jax.experimental.pallas import tpu_sc as plsc

N_EMBD = 64
SEQ = 200
BATCH = 4096

NC, NS = 2, 16
NW = NC * NS
TOTAL = BATCH * SEQ
RPW = TOTAL // NW
CHUNK = 400
NCHUNK = RPW // CHUNK
NSTREAM = 5
SPS = CHUNK // NSTREAM
LPR = N_EMBD // 16


def _body(x_hbm, pos_hbm, tok_hbm, out_hbm, idx_v, rows_v, pos_v, gsem):
    cid = lax.axis_index("c")
    sid = lax.axis_index("s")
    wid = sid * NC + cid
    pltpu.sync_copy(pos_hbm, pos_v)
    base = wid * RPW

    def chunk_body(c, carry):
        pltpu.sync_copy(x_hbm.at[wid * NCHUNK + c], idx_v)
        descs = []
        for j in range(NSTREAM):
            descs.append(pltpu.async_copy(
                tok_hbm.at[idx_v.at[j]],
                rows_v.at[pl.ds(j * SPS, SPS)], gsem))
        for d in descs:
            d.wait()

        def add_body(r, carry2):
            for k in range(LPR):
                plsc.addupdate(rows_v.at[r, pl.ds(k * 16, 16)],
                               pos_v[r, pl.ds(k * 16, 16)])
            return carry2

        lax.fori_loop(0, CHUNK, add_body, 0, unroll=4)
        pltpu.sync_copy(rows_v, out_hbm.at[pl.ds(base + c * CHUNK, CHUNK)])
        return carry

    lax.fori_loop(0, NCHUNK, chunk_body, 0)


def kernel(x, tok_emb, pos_emb):
    x2 = x.astype(jnp.int32).reshape(NW * NCHUNK, NSTREAM, SPS)
    pos_rep = jnp.concatenate([pos_emb[:SEQ]] * (CHUNK // SEQ), axis=0)
    mesh = plsc.VectorSubcoreMesh(core_axis_name="c", subcore_axis_name="s")
    f = pl.kernel(
        _body,
        out_type=jax.ShapeDtypeStruct((TOTAL, N_EMBD), jnp.float32),
        mesh=mesh,
        compiler_params=pltpu.CompilerParams(use_tc_tiling_on_sc=False),
        scratch_types=[
            pltpu.VMEM((NSTREAM, SPS), jnp.int32),
            pltpu.VMEM((CHUNK, N_EMBD), jnp.float32),
            pltpu.VMEM((CHUNK, N_EMBD), jnp.float32),
            pltpu.SemaphoreType.DMA,
        ],
    )
    out = f(x2, pos_rep, tok_emb)
    return out.reshape(BATCH, SEQ, N_EMBD)
